# baseline (speedup 1.0000x reference)
import functools

import jax
import jax.numpy as jnp
from jax import lax
from jax.experimental import pallas as pl
from jax.experimental.pallas import tpu as pltpu
from jax.experimental.pallas import tpu_sc as plsc

DIM = 128
BATCH = 16384
NCHUNKS = 2
CB = BATCH // NCHUNKS

_info = plsc.get_sparse_core_info()
NC, NS, L = _info.num_cores, _info.num_subcores, _info.num_lanes
NW = NC * NS
BPW = CB // NW
NGC = BPW // 128

_sc_mesh = plsc.VectorSubcoreMesh(core_axis_name="c", subcore_axis_name="s")


@functools.partial(
    pl.kernel,
    mesh=_sc_mesh,
    out_type=[jax.ShapeDtypeStruct((CB, DIM), jnp.float32) for _ in range(4)],
    scratch_types=[
        pltpu.VMEM((NGC, 128), jnp.int32),
        pltpu.VMEM((NGC, 128), jnp.int32),
        pltpu.VMEM((BPW, DIM), jnp.float32),
        pltpu.SemaphoreType.DMA,
    ],
)
def _sc_gather(uidx_hbm, iidx_hbm, tab_mu, tab_mi, tab_gu, tab_gi,
               out_mu, out_mi, out_gu, out_gi, idx_u, idx_i, rows, sem):
    wid = lax.axis_index("s") * NC + lax.axis_index("c")
    base = wid * BPW
    pltpu.sync_copy(uidx_hbm.at[wid], idx_u)
    pltpu.sync_copy(iidx_hbm.at[wid], idx_i)
    for table, idx, out in (
        (tab_mu, idx_u, out_mu),
        (tab_mi, idx_i, out_mi),
        (tab_gu, idx_u, out_gu),
        (tab_gi, idx_i, out_gi),
    ):
        handles = []
        for c in range(NGC):
            handles.append(
                pltpu.async_copy(table.at[idx.at[c]],
                                 rows.at[pl.ds(c * 128, 128)], sem))
        for h in handles:
            h.wait()
        pltpu.sync_copy(rows, out.at[pl.ds(base, BPW)])


_TC_BLK = 2048


def _tc_body(mu_ref, mi_ref, gu_ref, gi_ref, w1a_ref, w1b_ref, b1_ref,
             w2_ref, b2_ref, w3_ref, b3_ref, wg_ref, wm_ref, bl_ref, out_ref):
    f32 = jnp.float32
    h = jnp.dot(mu_ref[...], w1a_ref[...], preferred_element_type=f32)
    h += jnp.dot(mi_ref[...], w1b_ref[...], preferred_element_type=f32)
    h = jnp.maximum(h + b1_ref[...], 0.0)
    h = jnp.maximum(jnp.dot(h, w2_ref[...], preferred_element_type=f32)
                    + b2_ref[...], 0.0)
    h = jnp.maximum(jnp.dot(h, w3_ref[...], preferred_element_type=f32)
                    + b3_ref[...], 0.0)
    g = gu_ref[...] * gi_ref[...]
    out = jnp.dot(g, wg_ref[...], preferred_element_type=f32)
    out += jnp.dot(h, wm_ref[...], preferred_element_type=f32)
    out_ref[...] = out + bl_ref[...]


def _fixed(shape):
    return pl.BlockSpec(shape, lambda b: (0, 0))


_tc_dense = pl.pallas_call(
    _tc_body,
    grid=(CB // _TC_BLK,),
    in_specs=[
        pl.BlockSpec((_TC_BLK, DIM), lambda b: (b, 0)),
        pl.BlockSpec((_TC_BLK, DIM), lambda b: (b, 0)),
        pl.BlockSpec((_TC_BLK, DIM), lambda b: (b, 0)),
        pl.BlockSpec((_TC_BLK, DIM), lambda b: (b, 0)),
        _fixed((DIM, 64)),
        _fixed((DIM, 64)),
        _fixed((1, 64)),
        _fixed((64, 16)),
        _fixed((1, 16)),
        _fixed((16, 8)),
        _fixed((1, 8)),
        _fixed((DIM, 1)),
        _fixed((8, 1)),
        _fixed((1, 1)),
    ],
    out_specs=pl.BlockSpec((_TC_BLK, 1), lambda b: (b, 0)),
    out_shape=jax.ShapeDtypeStruct((CB, 1), jnp.float32),
)


def kernel(x, mlp_user_w, mlp_item_w, gmf_user_w, gmf_item_w,
           W1, b1, W2, b2, W3, b3, W_last, b_last):
    u = x[:, 0].astype(jnp.int32)
    i = x[:, 1].astype(jnp.int32)
    w1a = W1[:, :DIM].T
    w1b = W1[:, DIM:].T
    wg = W_last[0, :DIM].reshape(DIM, 1)
    wm = W_last[0, DIM:].reshape(8, 1)
    gathered = []
    for k in range(NCHUNKS):
        uu = lax.dynamic_slice_in_dim(u, k * CB, CB).reshape(NW, NGC, 128)
        ii = lax.dynamic_slice_in_dim(i, k * CB, CB).reshape(NW, NGC, 128)
        gathered.append(_sc_gather(uu, ii, mlp_user_w, mlp_item_w,
                                   gmf_user_w, gmf_item_w))
    outs = []
    for k in range(NCHUNKS):
        mu, mi, gu, gi = gathered[k]
        outs.append(_tc_dense(mu, mi, gu, gi, w1a, w1b, b1.reshape(1, 64),
                              W2.T, b2.reshape(1, 16), W3.T, b3.reshape(1, 8),
                              wg, wm, b_last.reshape(1, 1)))
    return jnp.concatenate(outs, axis=0)


# speedup vs baseline: 1.0410x; 1.0410x over previous
import functools

import jax
import jax.numpy as jnp
from jax import lax
from jax.experimental import pallas as pl
from jax.experimental.pallas import tpu as pltpu
from jax.experimental.pallas import tpu_sc as plsc

DIM = 128
BATCH = 16384

_info = plsc.get_sparse_core_info()
NC, NS, L = _info.num_cores, _info.num_subcores, _info.num_lanes
NW = NC * NS
BPW = BATCH // NW
NGC = BPW // 128
NSLOT = 6

_sc_mesh = plsc.VectorSubcoreMesh(core_axis_name="c", subcore_axis_name="s")


@functools.partial(
    pl.kernel,
    mesh=_sc_mesh,
    out_type=[jax.ShapeDtypeStruct((BATCH, DIM), jnp.float32) for _ in range(4)],
    scratch_types=[
        pltpu.VMEM((NGC, 128), jnp.int32),
        pltpu.VMEM((NGC, 128), jnp.int32),
        pltpu.VMEM((NSLOT * 128, DIM), jnp.float32),
        pltpu.SemaphoreType.DMA,
        pltpu.SemaphoreType.DMA,
    ],
)
def _sc_gather(uidx_hbm, iidx_hbm, tab_mu, tab_mi, tab_gu, tab_gi,
               out_mu, out_mi, out_gu, out_gi, idx_u, idx_i, bufs, gsem, wsem):
    wid = lax.axis_index("s") * NC + lax.axis_index("c")
    base = wid * BPW
    pltpu.sync_copy(uidx_hbm.at[wid], idx_u)
    pltpu.sync_copy(iidx_hbm.at[wid], idx_i)
    units = []
    for table, idx, out in (
        (tab_mu, idx_u, out_mu),
        (tab_mi, idx_i, out_mi),
        (tab_gu, idx_u, out_gu),
        (tab_gi, idx_i, out_gi),
    ):
        for c in range(NGC):
            units.append((table, idx.at[c],
                          out.at[pl.ds(base + c * 128, 128)]))
    T = len(units)
    gh = [None] * T
    wh = [None] * T

    def slot(t):
        return bufs.at[pl.ds((t % NSLOT) * 128, 128)]

    for t, (table, idxs, dst) in enumerate(units):
        if t >= NSLOT:
            wh[t - NSLOT].wait()
        gh[t] = pltpu.async_copy(table.at[idxs], slot(t), gsem)
        if t >= 1:
            gh[t - 1].wait()
            wh[t - 1] = pltpu.async_copy(slot(t - 1), units[t - 1][2], wsem)
    gh[T - 1].wait()
    wh[T - 1] = pltpu.async_copy(slot(T - 1), units[T - 1][2], wsem)
    for t in range(T - NSLOT, T):
        wh[t].wait()


_TC_BLK = 2048


def _tc_body(mu_ref, mi_ref, gu_ref, gi_ref, w1a_ref, w1b_ref, b1_ref,
             w2_ref, b2_ref, w3_ref, b3_ref, wg_ref, wm_ref, bl_ref, out_ref):
    f32 = jnp.float32
    h = jnp.dot(mu_ref[...], w1a_ref[...], preferred_element_type=f32)
    h += jnp.dot(mi_ref[...], w1b_ref[...], preferred_element_type=f32)
    h = jnp.maximum(h + b1_ref[...], 0.0)
    h = jnp.maximum(jnp.dot(h, w2_ref[...], preferred_element_type=f32)
                    + b2_ref[...], 0.0)
    h = jnp.maximum(jnp.dot(h, w3_ref[...], preferred_element_type=f32)
                    + b3_ref[...], 0.0)
    g = gu_ref[...] * gi_ref[...]
    out = jnp.dot(g, wg_ref[...], preferred_element_type=f32)
    out += jnp.dot(h, wm_ref[...], preferred_element_type=f32)
    out_ref[...] = out + bl_ref[...]


def _fixed(shape):
    return pl.BlockSpec(shape, lambda b: (0, 0))


_tc_dense = pl.pallas_call(
    _tc_body,
    grid=(BATCH // _TC_BLK,),
    in_specs=[
        pl.BlockSpec((_TC_BLK, DIM), lambda b: (b, 0)),
        pl.BlockSpec((_TC_BLK, DIM), lambda b: (b, 0)),
        pl.BlockSpec((_TC_BLK, DIM), lambda b: (b, 0)),
        pl.BlockSpec((_TC_BLK, DIM), lambda b: (b, 0)),
        _fixed((DIM, 64)),
        _fixed((DIM, 64)),
        _fixed((1, 64)),
        _fixed((64, 16)),
        _fixed((1, 16)),
        _fixed((16, 8)),
        _fixed((1, 8)),
        _fixed((DIM, 1)),
        _fixed((8, 1)),
        _fixed((1, 1)),
    ],
    out_specs=pl.BlockSpec((_TC_BLK, 1), lambda b: (b, 0)),
    out_shape=jax.ShapeDtypeStruct((BATCH, 1), jnp.float32),
)


def kernel(x, mlp_user_w, mlp_item_w, gmf_user_w, gmf_item_w,
           W1, b1, W2, b2, W3, b3, W_last, b_last):
    u = x[:, 0].astype(jnp.int32).reshape(NW, NGC, 128)
    i = x[:, 1].astype(jnp.int32).reshape(NW, NGC, 128)
    mu, mi, gu, gi = _sc_gather(u, i, mlp_user_w, mlp_item_w,
                                gmf_user_w, gmf_item_w)
    w1a = W1[:, :DIM].T
    w1b = W1[:, DIM:].T
    wg = W_last[0, :DIM].reshape(DIM, 1)
    wm = W_last[0, DIM:].reshape(8, 1)
    return _tc_dense(mu, mi, gu, gi, w1a, w1b, b1.reshape(1, 64),
                     W2.T, b2.reshape(1, 16), W3.T, b3.reshape(1, 8),
                     wg, wm, b_last.reshape(1, 1))


# speedup vs baseline: 1.0856x; 1.0429x over previous
import functools

import jax
import jax.numpy as jnp
from jax import lax
from jax.experimental import pallas as pl
from jax.experimental.pallas import tpu as pltpu
from jax.experimental.pallas import tpu_sc as plsc

DIM = 128
BATCH = 16384

_info = plsc.get_sparse_core_info()
NC, NS, L = _info.num_cores, _info.num_subcores, _info.num_lanes
NW = NC * NS
BPW = BATCH // NW
NGC = BPW // 128
NSLOT = 6

_sc_mesh = plsc.VectorSubcoreMesh(core_axis_name="c", subcore_axis_name="s")


@functools.partial(
    pl.kernel,
    mesh=_sc_mesh,
    out_type=[jax.ShapeDtypeStruct((BATCH, DIM), jnp.float32) for _ in range(3)],
    scratch_types=[
        pltpu.VMEM((NGC, 128), jnp.int32),
        pltpu.VMEM((NGC, 128), jnp.int32),
        pltpu.VMEM((NSLOT * 128, DIM), jnp.float32),
        pltpu.SemaphoreType.DMA,
        pltpu.SemaphoreType.DMA,
    ],
)
def _sc_gather(uidx_hbm, iidx_hbm, tab_mu, tab_mi, tab_gu, tab_gi,
               out_mu, out_mi, out_g, idx_u, idx_i, bufs, gsem, wsem):
    wid = lax.axis_index("s") * NC + lax.axis_index("c")
    base = wid * BPW

    def slot(s):
        return bufs.at[pl.ds((s % NSLOT) * 128, 128)]

    pltpu.sync_copy(uidx_hbm.at[wid], idx_u)
    pltpu.sync_copy(iidx_hbm.at[wid], idx_i)

    units = []
    for table, idx, out in ((tab_mu, idx_u, out_mu), (tab_mi, idx_i, out_mi)):
        for c in range(NGC):
            units.append((table, idx.at[c],
                          out.at[pl.ds(base + c * 128, 128)]))
    T = len(units)
    gh = [None] * T
    wh = [None] * T
    w_waited = [False] * T
    for t, (table, idxs, dst) in enumerate(units):
        if t >= NSLOT:
            wh[t - NSLOT].wait()
            w_waited[t - NSLOT] = True
        gh[t] = pltpu.async_copy(table.at[idxs], slot(t), gsem)
        if t >= 1:
            gh[t - 1].wait()
            wh[t - 1] = pltpu.async_copy(slot(t - 1), units[t - 1][2], wsem)
    gh[T - 1].wait()
    wh[T - 1] = pltpu.async_copy(slot(T - 1), units[T - 1][2], wsem)

    mlp_unit_for_slot = {0: 6, 1: 7, 2: 2, 3: 3, 4: 4, 5: 5}
    pg = [None] * NGC
    qg = [None] * NGC
    pw = [None] * NGC
    p_waited = [False] * NGC

    def fire(c):
        for s in ((2 * c) % NSLOT, (2 * c + 1) % NSLOT):
            t = mlp_unit_for_slot[s]
            if not w_waited[t]:
                wh[t].wait()
                w_waited[t] = True
        if c >= 3:
            pw[c - 3].wait()
            p_waited[c - 3] = True
        pg[c] = pltpu.async_copy(tab_gu.at[idx_u.at[c]], slot(2 * c), gsem)
        qg[c] = pltpu.async_copy(tab_gi.at[idx_i.at[c]], slot(2 * c + 1),
                                 gsem)

    for c in range(min(3, NGC)):
        fire(c)
    for c in range(NGC):
        pg[c].wait()
        qg[c].wait()
        a = slot(2 * c)
        b = slot(2 * c + 1)

        def body(r, _):
            for j in range(DIM // 16):
                sl = pl.ds(j * 16, 16)
                a[r, sl] = a[r, sl] * b[r, sl]
            return 0

        lax.fori_loop(0, 128, body, 0)
        pw[c] = pltpu.async_copy(a, out_g.at[pl.ds(base + c * 128, 128)],
                                 wsem)
        if c + 3 < NGC:
            fire(c + 3)
    for t in range(T):
        if not w_waited[t]:
            wh[t].wait()
    for c in range(NGC):
        if not p_waited[c]:
            pw[c].wait()


_TC_BLK = 2048


def _tc_body(mu_ref, mi_ref, g_ref, w1a_ref, w1b_ref, b1_ref,
             w2_ref, b2_ref, w3_ref, b3_ref, wg_ref, wm_ref, bl_ref, out_ref):
    f32 = jnp.float32
    h = jnp.dot(mu_ref[...], w1a_ref[...], preferred_element_type=f32)
    h += jnp.dot(mi_ref[...], w1b_ref[...], preferred_element_type=f32)
    h = jnp.maximum(h + b1_ref[...], 0.0)
    h = jnp.maximum(jnp.dot(h, w2_ref[...], preferred_element_type=f32)
                    + b2_ref[...], 0.0)
    h = jnp.maximum(jnp.dot(h, w3_ref[...], preferred_element_type=f32)
                    + b3_ref[...], 0.0)
    out = jnp.dot(g_ref[...], wg_ref[...], preferred_element_type=f32)
    out += jnp.dot(h, wm_ref[...], preferred_element_type=f32)
    out_ref[...] = out + bl_ref[...]


def _fixed(shape):
    return pl.BlockSpec(shape, lambda b: (0, 0))


_tc_dense = pl.pallas_call(
    _tc_body,
    grid=(BATCH // _TC_BLK,),
    in_specs=[
        pl.BlockSpec((_TC_BLK, DIM), lambda b: (b, 0)),
        pl.BlockSpec((_TC_BLK, DIM), lambda b: (b, 0)),
        pl.BlockSpec((_TC_BLK, DIM), lambda b: (b, 0)),
        _fixed((DIM, 64)),
        _fixed((DIM, 64)),
        _fixed((1, 64)),
        _fixed((64, 16)),
        _fixed((1, 16)),
        _fixed((16, 8)),
        _fixed((1, 8)),
        _fixed((DIM, 1)),
        _fixed((8, 1)),
        _fixed((1, 1)),
    ],
    out_specs=pl.BlockSpec((_TC_BLK, 1), lambda b: (b, 0)),
    out_shape=jax.ShapeDtypeStruct((BATCH, 1), jnp.float32),
)


def kernel(x, mlp_user_w, mlp_item_w, gmf_user_w, gmf_item_w,
           W1, b1, W2, b2, W3, b3, W_last, b_last):
    u = x[:, 0].astype(jnp.int32).reshape(NW, NGC, 128)
    i = x[:, 1].astype(jnp.int32).reshape(NW, NGC, 128)
    mu, mi, g = _sc_gather(u, i, mlp_user_w, mlp_item_w,
                           gmf_user_w, gmf_item_w)
    w1a = W1[:, :DIM].T
    w1b = W1[:, DIM:].T
    wg = W_last[0, :DIM].reshape(DIM, 1)
    wm = W_last[0, DIM:].reshape(8, 1)
    return _tc_dense(mu, mi, g, w1a, w1b, b1.reshape(1, 64),
                     W2.T, b2.reshape(1, 16), W3.T, b3.reshape(1, 8),
                     wg, wm, b_last.reshape(1, 1))
